# baseline (speedup 1.0000x reference)
import functools

import jax
import jax.numpy as jnp
from jax.experimental import pallas as pl
from jax.experimental.pallas import tpu as pltpu

D_IN = 768
D_SAE = 24576
TOPK = 64


def _encode_kernel(x_ref, w_ref, benc_ref, bdec_ref, out_ref):
    xc = x_ref[...] - bdec_ref[...]
    out_ref[...] = (
        jnp.dot(xc, w_ref[...], preferred_element_type=jnp.float32)
        + benc_ref[...]
    )


def _encode(x, w_enc, b_enc, b_dec, tm=256, tn=2048):
    t = x.shape[0]
    grid = (t // tm, D_SAE // tn)
    return pl.pallas_call(
        _encode_kernel,
        grid=grid,
        in_specs=[
            pl.BlockSpec((tm, D_IN), lambda i, j: (i, 0)),
            pl.BlockSpec((D_IN, tn), lambda i, j: (0, j)),
            pl.BlockSpec((1, tn), lambda i, j: (0, j)),
            pl.BlockSpec((1, D_IN), lambda i, j: (0, 0)),
        ],
        out_specs=pl.BlockSpec((tm, tn), lambda i, j: (i, j)),
        out_shape=jax.ShapeDtypeStruct((t, D_SAE), jnp.float32),
    )(x, w_enc, b_enc.reshape(1, -1), b_dec.reshape(1, -1))


def _ceil_avg(lo, hi):
    return (lo >> 1) + (hi >> 1) + (lo & hi & 1) + ((lo ^ hi) & 1)


def _topk_mask_kernel(h_ref, out_ref, keys_ref, lo_ref, hi_ref, n_ref):
    tb = h_ref.shape[0]
    h = h_ref[...]
    s = jax.lax.bitcast_convert_type(h, jnp.int32)
    keys = jnp.where(s < 0, s ^ jnp.int32(0x7FFFFFFF), s)
    keys_ref[...] = keys

    cm = jnp.max(keys.reshape(tb, D_SAE // 128, 128), axis=2)
    lo_a = jnp.min(cm, axis=1, keepdims=True)
    hi_a = jnp.max(cm, axis=1, keepdims=True)
    row_max = hi_a

    def body_a(_, carry):
        lo, hi = carry
        mid = _ceil_avg(lo, hi)
        cnt = jnp.sum((cm >= mid).astype(jnp.int32), axis=1, keepdims=True)
        ok = cnt >= TOPK
        return jnp.where(ok, mid, lo), jnp.where(ok, hi, mid - 1)

    m64, _ = jax.lax.fori_loop(0, 32, body_a, (lo_a, hi_a))
    lo_ref[...] = m64
    hi_ref[...] = row_max

    w = jnp.maximum(row_max.astype(jnp.float32) - m64.astype(jnp.float32),
                    1.0)
    e = (jax.lax.bitcast_convert_type(w, jnp.int32) >> 23) - 127
    n_ref[0] = jnp.minimum(jnp.max(e) + 3, 32)

    def body_b(_, carry):
        lo, hi = carry
        mid = _ceil_avg(lo, hi)
        cnt = jnp.sum((keys_ref[...] >= mid).astype(jnp.int32), axis=1,
                      keepdims=True)
        ok = cnt >= TOPK
        return jnp.where(ok, mid, lo), jnp.where(ok, hi, mid - 1)

    lo_b, _ = jax.lax.fori_loop(0, 32, body_b, (m64, row_max))
    mask = keys_ref[...] >= lo_b
    out_ref[...] = jnp.where(mask, jnp.maximum(h_ref[...], 0.0), 0.0)


def _topk_mask(h, tb=64):
    t = h.shape[0]
    return pl.pallas_call(
        _topk_mask_kernel,
        grid=(t // tb,),
        in_specs=[pl.BlockSpec((tb, D_SAE), lambda i: (i, 0))],
        out_specs=pl.BlockSpec((tb, D_SAE), lambda i: (i, 0)),
        out_shape=jax.ShapeDtypeStruct((t, D_SAE), jnp.float32),
        scratch_shapes=[
            pltpu.VMEM((tb, D_SAE), jnp.int32),
            pltpu.VMEM((tb, 1), jnp.int32),
            pltpu.VMEM((tb, 1), jnp.int32),
            pltpu.SMEM((1,), jnp.int32),
        ],
    )(h)


def _decode_kernel(hs_ref, w_ref, bdec_ref, out_ref):
    j = pl.program_id(1)

    @pl.when(j == 0)
    def _():
        out_ref[...] = jnp.broadcast_to(bdec_ref[...], out_ref.shape)

    out_ref[...] += jnp.dot(hs_ref[...], w_ref[...],
                            preferred_element_type=jnp.float32)


def _decode(h_sparse, w_dec, b_dec, tm=2048, kb=512):
    t = h_sparse.shape[0]
    grid = (t // tm, D_SAE // kb)
    return pl.pallas_call(
        _decode_kernel,
        grid=grid,
        in_specs=[
            pl.BlockSpec((tm, kb), lambda i, j: (i, j)),
            pl.BlockSpec((kb, D_IN), lambda i, j: (j, 0)),
            pl.BlockSpec((1, D_IN), lambda i, j: (0, 0)),
        ],
        out_specs=pl.BlockSpec((tm, D_IN), lambda i, j: (i, 0)),
        out_shape=jax.ShapeDtypeStruct((t, D_IN), jnp.float32),
    )(h_sparse, w_dec, b_dec.reshape(1, -1))


@jax.jit
def kernel(x, W_enc, b_enc, W_dec, b_dec):
    h = _encode(x, W_enc, b_enc, b_dec)
    h_sparse = _topk_mask(h)
    x_hat = _decode(h_sparse, W_dec, b_dec)
    return (x_hat, h_sparse)


# speedup vs baseline: 2.5041x; 2.5041x over previous
import functools

import jax
import jax.numpy as jnp
from jax.experimental import pallas as pl
from jax.experimental.pallas import tpu as pltpu

D_IN = 768
D_SAE = 24576
TOPK = 64


def _encode_kernel(x_ref, w_ref, benc_ref, bdec_ref, out_ref):
    xc = x_ref[...] - bdec_ref[...]
    out_ref[...] = (
        jnp.dot(xc, w_ref[...], preferred_element_type=jnp.float32)
        + benc_ref[...]
    )


def _encode(x, w_enc, b_enc, b_dec, tm=256, tn=2048):
    t = x.shape[0]
    grid = (t // tm, D_SAE // tn)
    return pl.pallas_call(
        _encode_kernel,
        grid=grid,
        in_specs=[
            pl.BlockSpec((tm, D_IN), lambda i, j: (i, 0)),
            pl.BlockSpec((D_IN, tn), lambda i, j: (0, j)),
            pl.BlockSpec((1, tn), lambda i, j: (0, j)),
            pl.BlockSpec((1, D_IN), lambda i, j: (0, 0)),
        ],
        out_specs=pl.BlockSpec((tm, tn), lambda i, j: (i, j)),
        out_shape=jax.ShapeDtypeStruct((t, D_SAE), jnp.float32),
    )(x, w_enc, b_enc.reshape(1, -1), b_dec.reshape(1, -1))


def _ceil_avg(lo, hi):
    return (lo >> 1) + (hi >> 1) + (lo & hi & 1) + ((lo ^ hi) & 1)


def _topk_mask_kernel(h_ref, out_ref, keys_ref):
    h = h_ref[...]
    s = jax.lax.bitcast_convert_type(h, jnp.int32)
    keys = jnp.where(s < 0, s ^ jnp.int32(0x7FFFFFFF), s)
    keys_ref[...] = keys

    lo = jnp.min(keys, axis=1, keepdims=True)
    hi = jnp.max(keys, axis=1, keepdims=True)

    def body(_, carry):
        lo, hi = carry
        mid = _ceil_avg(lo, hi)
        cnt = jnp.sum((keys_ref[...] >= mid).astype(jnp.int32), axis=1,
                      keepdims=True)
        ok = cnt >= TOPK
        return jnp.where(ok, mid, lo), jnp.where(ok, hi, mid - 1)

    lo, hi = jax.lax.fori_loop(0, 32, body, (lo, hi))
    mask = keys_ref[...] >= lo
    out_ref[...] = jnp.where(mask, jnp.maximum(h_ref[...], 0.0), 0.0)


def _topk_mask(h, tb=64):
    t = h.shape[0]
    return pl.pallas_call(
        _topk_mask_kernel,
        grid=(t // tb,),
        in_specs=[pl.BlockSpec((tb, D_SAE), lambda i: (i, 0))],
        out_specs=pl.BlockSpec((tb, D_SAE), lambda i: (i, 0)),
        out_shape=jax.ShapeDtypeStruct((t, D_SAE), jnp.float32),
        scratch_shapes=[pltpu.VMEM((tb, D_SAE), jnp.int32)],
    )(h)


def _decode_kernel(hs_ref, w_ref, bdec_ref, out_ref):
    j = pl.program_id(1)

    @pl.when(j == 0)
    def _():
        out_ref[...] = jnp.broadcast_to(bdec_ref[...], out_ref.shape)

    out_ref[...] += jnp.dot(hs_ref[...], w_ref[...],
                            preferred_element_type=jnp.float32)


def _decode(h_sparse, w_dec, b_dec, tm=2048, kb=512):
    t = h_sparse.shape[0]
    grid = (t // tm, D_SAE // kb)
    return pl.pallas_call(
        _decode_kernel,
        grid=grid,
        in_specs=[
            pl.BlockSpec((tm, kb), lambda i, j: (i, j)),
            pl.BlockSpec((kb, D_IN), lambda i, j: (j, 0)),
            pl.BlockSpec((1, D_IN), lambda i, j: (0, 0)),
        ],
        out_specs=pl.BlockSpec((tm, D_IN), lambda i, j: (i, 0)),
        out_shape=jax.ShapeDtypeStruct((t, D_IN), jnp.float32),
    )(h_sparse, w_dec, b_dec.reshape(1, -1))


@jax.jit
def kernel(x, W_enc, b_enc, W_dec, b_dec):
    h = _encode(x, W_enc, b_enc, b_dec)
    h_sparse = _topk_mask(h)
    x_hat = _decode(h_sparse, W_dec, b_dec)
    return (x_hat, h_sparse)


# speedup vs baseline: 2.6607x; 1.0625x over previous
import functools

import jax
import jax.numpy as jnp
from jax.experimental import pallas as pl
from jax.experimental.pallas import tpu as pltpu

D_IN = 768
D_SAE = 24576
TOPK = 64


def _encode_kernel(x_ref, w_ref, benc_ref, bdec_ref, out_ref):
    xc = x_ref[...] - bdec_ref[...]
    out_ref[...] = (
        jnp.dot(xc, w_ref[...], preferred_element_type=jnp.float32)
        + benc_ref[...]
    )


def _encode(x, w_enc, b_enc, b_dec, tm=256, tn=2048):
    t = x.shape[0]
    grid = (t // tm, D_SAE // tn)
    return pl.pallas_call(
        _encode_kernel,
        grid=grid,
        in_specs=[
            pl.BlockSpec((tm, D_IN), lambda i, j: (i, 0)),
            pl.BlockSpec((D_IN, tn), lambda i, j: (0, j)),
            pl.BlockSpec((1, tn), lambda i, j: (0, j)),
            pl.BlockSpec((1, D_IN), lambda i, j: (0, 0)),
        ],
        out_specs=pl.BlockSpec((tm, tn), lambda i, j: (i, j)),
        out_shape=jax.ShapeDtypeStruct((t, D_SAE), jnp.float32),
    )(x, w_enc, b_enc.reshape(1, -1), b_dec.reshape(1, -1))


def _ceil_avg(lo, hi):
    return (lo >> 1) + (hi >> 1) + (lo & hi & 1) + ((lo ^ hi) & 1)


def _topk_mask_kernel(h_ref, out_ref, keys_ref):
    h = h_ref[...]
    s = jax.lax.bitcast_convert_type(h, jnp.int32)
    keys = jnp.where(s < 0, s ^ jnp.int32(0x7FFFFFFF), s)
    keys_ref[...] = keys

    lo = jnp.min(keys, axis=1, keepdims=True)
    hi = jnp.max(keys, axis=1, keepdims=True)

    def body(_, carry):
        lo, hi = carry
        mid = _ceil_avg(lo, hi)
        cnt = jnp.sum((keys_ref[...] >= mid).astype(jnp.int32), axis=1,
                      keepdims=True)
        ok = cnt >= TOPK
        return jnp.where(ok, mid, lo), jnp.where(ok, hi, mid - 1)

    lo, hi = jax.lax.fori_loop(0, 32, body, (lo, hi))
    mask = keys_ref[...] >= lo
    out_ref[...] = jnp.where(mask, jnp.maximum(h_ref[...], 0.0), 0.0)


def _topk_mask(h, tb=128):
    t = h.shape[0]
    return pl.pallas_call(
        _topk_mask_kernel,
        grid=(t // tb,),
        in_specs=[pl.BlockSpec((tb, D_SAE), lambda i: (i, 0))],
        out_specs=pl.BlockSpec((tb, D_SAE), lambda i: (i, 0)),
        out_shape=jax.ShapeDtypeStruct((t, D_SAE), jnp.float32),
        scratch_shapes=[pltpu.VMEM((tb, D_SAE), jnp.int32)],
        compiler_params=pltpu.CompilerParams(
            vmem_limit_bytes=100 * 1024 * 1024),
    )(h)


def _decode_kernel(hs_ref, w_ref, bdec_ref, out_ref):
    j = pl.program_id(1)

    @pl.when(j == 0)
    def _():
        out_ref[...] = jnp.broadcast_to(bdec_ref[...], out_ref.shape)

    out_ref[...] += jnp.dot(hs_ref[...], w_ref[...],
                            preferred_element_type=jnp.float32)


def _decode(h_sparse, w_dec, b_dec, tm=2048, kb=512):
    t = h_sparse.shape[0]
    grid = (t // tm, D_SAE // kb)
    return pl.pallas_call(
        _decode_kernel,
        grid=grid,
        in_specs=[
            pl.BlockSpec((tm, kb), lambda i, j: (i, j)),
            pl.BlockSpec((kb, D_IN), lambda i, j: (j, 0)),
            pl.BlockSpec((1, D_IN), lambda i, j: (0, 0)),
        ],
        out_specs=pl.BlockSpec((tm, D_IN), lambda i, j: (i, 0)),
        out_shape=jax.ShapeDtypeStruct((t, D_IN), jnp.float32),
    )(h_sparse, w_dec, b_dec.reshape(1, -1))


@jax.jit
def kernel(x, W_enc, b_enc, W_dec, b_dec):
    h = _encode(x, W_enc, b_enc, b_dec)
    h_sparse = _topk_mask(h)
    x_hat = _decode(h_sparse, W_dec, b_dec)
    return (x_hat, h_sparse)


# speedup vs baseline: 2.6672x; 1.0024x over previous
import jax
import jax.numpy as jnp
from jax.experimental import pallas as pl
from jax.experimental.pallas import tpu as pltpu

D_IN = 768
D_SAE = 24576
TOPK = 64


def _encode_kernel(x_ref, w_ref, benc_ref, bdec_ref, out_ref):
    xc = x_ref[...] - bdec_ref[...]
    out_ref[...] = (
        jnp.dot(xc, w_ref[...], preferred_element_type=jnp.float32)
        + benc_ref[...]
    )


def _encode(x, w_enc, b_enc, b_dec, tm=256, tn=2048):
    t = x.shape[0]
    grid = (t // tm, D_SAE // tn)
    return pl.pallas_call(
        _encode_kernel,
        grid=grid,
        in_specs=[
            pl.BlockSpec((tm, D_IN), lambda i, j: (i, 0)),
            pl.BlockSpec((D_IN, tn), lambda i, j: (0, j)),
            pl.BlockSpec((1, tn), lambda i, j: (0, j)),
            pl.BlockSpec((1, D_IN), lambda i, j: (0, 0)),
        ],
        out_specs=pl.BlockSpec((tm, tn), lambda i, j: (i, j)),
        out_shape=jax.ShapeDtypeStruct((t, D_SAE), jnp.float32),
    )(x, w_enc, b_enc.reshape(1, -1), b_dec.reshape(1, -1))


def _ceil_avg(lo, hi):
    return (lo >> 1) + (hi >> 1) + (lo & hi & 1) + ((lo ^ hi) & 1)


def _topk_mask_kernel(h_ref, out_ref, keys_ref):
    h = h_ref[...]
    s = jax.lax.bitcast_convert_type(h, jnp.int32)
    keys = jnp.where(s < 0, s ^ jnp.int32(0x7FFFFFFF), s)
    keys_ref[...] = keys

    lo = jnp.min(keys, axis=1, keepdims=True)
    hi = jnp.max(keys, axis=1, keepdims=True)

    def body(_, carry):
        lo, hi = carry
        mid = _ceil_avg(lo, hi)
        cnt = jnp.sum((keys_ref[...] >= mid).astype(jnp.int32), axis=1,
                      keepdims=True)
        ok = cnt >= TOPK
        return jnp.where(ok, mid, lo), jnp.where(ok, hi, mid - 1)

    lo, hi = jax.lax.fori_loop(0, 32, body, (lo, hi))
    mask = keys_ref[...] >= lo
    out_ref[...] = jnp.where(mask, jnp.maximum(h_ref[...], 0.0), 0.0)


def _topk_mask(h, tb=128):
    t = h.shape[0]
    return pl.pallas_call(
        _topk_mask_kernel,
        grid=(t // tb,),
        in_specs=[pl.BlockSpec((tb, D_SAE), lambda i: (i, 0))],
        out_specs=pl.BlockSpec((tb, D_SAE), lambda i: (i, 0)),
        out_shape=jax.ShapeDtypeStruct((t, D_SAE), jnp.float32),
        scratch_shapes=[pltpu.VMEM((tb, D_SAE), jnp.int32)],
        compiler_params=pltpu.CompilerParams(
            vmem_limit_bytes=100 * 1024 * 1024),
    )(h)


def _decode_kernel(hs_ref, w_ref, bdec_ref, out_ref):
    j = pl.program_id(1)

    @pl.when(j == 0)
    def _():
        out_ref[...] = jnp.broadcast_to(bdec_ref[...], out_ref.shape)

    out_ref[...] += jnp.dot(hs_ref[...], w_ref[...],
                            preferred_element_type=jnp.float32)


def _decode(h_sparse, w_dec, b_dec, tm=2048, kb=512):
    t = h_sparse.shape[0]
    grid = (t // tm, D_SAE // kb)
    return pl.pallas_call(
        _decode_kernel,
        grid=grid,
        in_specs=[
            pl.BlockSpec((tm, kb), lambda i, j: (i, j)),
            pl.BlockSpec((kb, D_IN), lambda i, j: (j, 0)),
            pl.BlockSpec((1, D_IN), lambda i, j: (0, 0)),
        ],
        out_specs=pl.BlockSpec((tm, D_IN), lambda i, j: (i, 0)),
        out_shape=jax.ShapeDtypeStruct((t, D_IN), jnp.float32),
    )(h_sparse, w_dec, b_dec.reshape(1, -1))


@jax.jit
def kernel(x, W_enc, b_enc, W_dec, b_dec):
    h = _encode(x, W_enc, b_enc, b_dec)
    h_sparse = _topk_mask(h)
    x_hat = _decode(h_sparse, W_dec, b_dec)
    return (x_hat, h_sparse)


# speedup vs baseline: 2.7115x; 1.0166x over previous
import jax
import jax.numpy as jnp
from jax.experimental import pallas as pl
from jax.experimental.pallas import tpu as pltpu

D_IN = 768
D_SAE = 24576
TOPK = 64


def _encode_kernel(x_ref, w_ref, benc_ref, bdec_ref, out_ref):
    xc = x_ref[...] - bdec_ref[...]
    out_ref[...] = (
        jnp.dot(xc, w_ref[...], preferred_element_type=jnp.float32)
        + benc_ref[...]
    )


def _encode(x, w_enc, b_enc, b_dec, tm=256, tn=4096):
    t = x.shape[0]
    grid = (t // tm, D_SAE // tn)
    return pl.pallas_call(
        _encode_kernel,
        grid=grid,
        in_specs=[
            pl.BlockSpec((tm, D_IN), lambda i, j: (i, 0)),
            pl.BlockSpec((D_IN, tn), lambda i, j: (0, j)),
            pl.BlockSpec((1, tn), lambda i, j: (0, j)),
            pl.BlockSpec((1, D_IN), lambda i, j: (0, 0)),
        ],
        out_specs=pl.BlockSpec((tm, tn), lambda i, j: (i, j)),
        out_shape=jax.ShapeDtypeStruct((t, D_SAE), jnp.float32),
    )(x, w_enc, b_enc.reshape(1, -1), b_dec.reshape(1, -1))


def _ceil_avg(lo, hi):
    return (lo >> 1) + (hi >> 1) + (lo & hi & 1) + ((lo ^ hi) & 1)


def _topk_mask_kernel(h_ref, out_ref, keys_ref):
    h = h_ref[...]
    s = jax.lax.bitcast_convert_type(h, jnp.int32)
    keys = jnp.where(s < 0, s ^ jnp.int32(0x7FFFFFFF), s)
    keys_ref[...] = keys

    lo = jnp.min(keys, axis=1, keepdims=True)
    hi = jnp.max(keys, axis=1, keepdims=True)

    def body(_, carry):
        lo, hi = carry
        mid = _ceil_avg(lo, hi)
        cnt = jnp.sum((keys_ref[...] >= mid).astype(jnp.int32), axis=1,
                      keepdims=True)
        ok = cnt >= TOPK
        return jnp.where(ok, mid, lo), jnp.where(ok, hi, mid - 1)

    lo, hi = jax.lax.fori_loop(0, 32, body, (lo, hi))
    mask = keys_ref[...] >= lo
    out_ref[...] = jnp.where(mask, jnp.maximum(h_ref[...], 0.0), 0.0)


def _topk_mask(h, tb=128):
    t = h.shape[0]
    return pl.pallas_call(
        _topk_mask_kernel,
        grid=(t // tb,),
        in_specs=[pl.BlockSpec((tb, D_SAE), lambda i: (i, 0))],
        out_specs=pl.BlockSpec((tb, D_SAE), lambda i: (i, 0)),
        out_shape=jax.ShapeDtypeStruct((t, D_SAE), jnp.float32),
        scratch_shapes=[pltpu.VMEM((tb, D_SAE), jnp.int32)],
        compiler_params=pltpu.CompilerParams(
            vmem_limit_bytes=100 * 1024 * 1024),
    )(h)


def _decode_kernel(hs_ref, w_ref, bdec_ref, out_ref):
    j = pl.program_id(1)

    @pl.when(j == 0)
    def _():
        out_ref[...] = jnp.broadcast_to(bdec_ref[...], out_ref.shape)

    out_ref[...] += jnp.dot(hs_ref[...], w_ref[...],
                            preferred_element_type=jnp.float32)


def _decode(h_sparse, w_dec, b_dec, tm=2048, kb=1024):
    t = h_sparse.shape[0]
    grid = (t // tm, D_SAE // kb)
    return pl.pallas_call(
        _decode_kernel,
        grid=grid,
        in_specs=[
            pl.BlockSpec((tm, kb), lambda i, j: (i, j)),
            pl.BlockSpec((kb, D_IN), lambda i, j: (j, 0)),
            pl.BlockSpec((1, D_IN), lambda i, j: (0, 0)),
        ],
        out_specs=pl.BlockSpec((tm, D_IN), lambda i, j: (i, 0)),
        out_shape=jax.ShapeDtypeStruct((t, D_IN), jnp.float32),
    )(h_sparse, w_dec, b_dec.reshape(1, -1))


@jax.jit
def kernel(x, W_enc, b_enc, W_dec, b_dec):
    h = _encode(x, W_enc, b_enc, b_dec)
    h_sparse = _topk_mask(h)
    x_hat = _decode(h_sparse, W_dec, b_dec)
    return (x_hat, h_sparse)


# speedup vs baseline: 2.9115x; 1.0738x over previous
import jax
import jax.numpy as jnp
from jax.experimental import pallas as pl
from jax.experimental.pallas import tpu as pltpu

D_IN = 768
D_SAE = 24576
TOPK = 64


def _encode_kernel(x_ref, w_ref, benc_ref, bdec_ref, out_ref):
    xc = x_ref[...] - bdec_ref[...]
    out_ref[...] = (
        jnp.dot(xc, w_ref[...], preferred_element_type=jnp.float32)
        + benc_ref[...]
    )


def _encode(x, w_enc, b_enc, b_dec, tm=512, tn=4096):
    t = x.shape[0]
    grid = (t // tm, D_SAE // tn)
    return pl.pallas_call(
        _encode_kernel,
        grid=grid,
        in_specs=[
            pl.BlockSpec((tm, D_IN), lambda i, j: (i, 0)),
            pl.BlockSpec((D_IN, tn), lambda i, j: (0, j)),
            pl.BlockSpec((1, tn), lambda i, j: (0, j)),
            pl.BlockSpec((1, D_IN), lambda i, j: (0, 0)),
        ],
        out_specs=pl.BlockSpec((tm, tn), lambda i, j: (i, j)),
        out_shape=jax.ShapeDtypeStruct((t, D_SAE), jnp.float32),
    )(x, w_enc, b_enc.reshape(1, -1), b_dec.reshape(1, -1))


def _ceil_avg(lo, hi):
    return (lo >> 1) + (hi >> 1) + (lo & hi & 1) + ((lo ^ hi) & 1)


def _topk_mask_kernel(h_ref, out_ref, keys_ref):
    h = h_ref[...]
    s = jax.lax.bitcast_convert_type(h, jnp.int32)
    keys = jnp.where(s < 0, s ^ jnp.int32(0x7FFFFFFF), s)
    keys_ref[...] = keys

    lo = jnp.min(keys, axis=1, keepdims=True)
    hi = jnp.max(keys, axis=1, keepdims=True)

    def body(_, carry):
        lo, hi = carry
        mid = _ceil_avg(lo, hi)
        cnt = jnp.sum((keys_ref[...] >= mid).astype(jnp.int32), axis=1,
                      keepdims=True)
        ok = cnt >= TOPK
        return jnp.where(ok, mid, lo), jnp.where(ok, hi, mid - 1)

    lo, hi = jax.lax.fori_loop(0, 32, body, (lo, hi))
    mask = keys_ref[...] >= lo
    out_ref[...] = jnp.where(mask, jnp.maximum(h_ref[...], 0.0), 0.0)


def _topk_mask(h, tb=128):
    t = h.shape[0]
    return pl.pallas_call(
        _topk_mask_kernel,
        grid=(t // tb,),
        in_specs=[pl.BlockSpec((tb, D_SAE), lambda i: (i, 0))],
        out_specs=pl.BlockSpec((tb, D_SAE), lambda i: (i, 0)),
        out_shape=jax.ShapeDtypeStruct((t, D_SAE), jnp.float32),
        scratch_shapes=[pltpu.VMEM((tb, D_SAE), jnp.int32)],
        compiler_params=pltpu.CompilerParams(
            vmem_limit_bytes=100 * 1024 * 1024),
    )(h)


def _decode_kernel(hs_ref, w_ref, bdec_ref, out_ref):
    j = pl.program_id(1)

    @pl.when(j == 0)
    def _():
        out_ref[...] = jnp.broadcast_to(bdec_ref[...], out_ref.shape)

    out_ref[...] += jnp.dot(hs_ref[...], w_ref[...],
                            preferred_element_type=jnp.float32)


def _decode(h_sparse, w_dec, b_dec, tm=2048, kb=1024):
    t = h_sparse.shape[0]
    grid = (t // tm, D_SAE // kb)
    return pl.pallas_call(
        _decode_kernel,
        grid=grid,
        in_specs=[
            pl.BlockSpec((tm, kb), lambda i, j: (i, j)),
            pl.BlockSpec((kb, D_IN), lambda i, j: (j, 0)),
            pl.BlockSpec((1, D_IN), lambda i, j: (0, 0)),
        ],
        out_specs=pl.BlockSpec((tm, D_IN), lambda i, j: (i, 0)),
        out_shape=jax.ShapeDtypeStruct((t, D_IN), jnp.float32),
    )(h_sparse, w_dec, b_dec.reshape(1, -1))


@jax.jit
def kernel(x, W_enc, b_enc, W_dec, b_dec):
    h = _encode(x, W_enc, b_enc, b_dec)
    h_sparse = _topk_mask(h)
    x_hat = _decode(h_sparse, W_dec, b_dec)
    return (x_hat, h_sparse)
